# baseline (speedup 1.0000x reference)
import dataclasses
import functools

import jax
import jax.numpy as jnp
from jax import lax
from jax.experimental import pallas as pl
from jax.experimental.pallas import tpu as pltpu
from jax.experimental.pallas import tpu_sc as plsc

E = 100000
ND = 512
NSM = 128
NB = 64
NSEG = 25000
NTGT = 25000
MEMR = 40000

NC = 2
NS = 16
NW = NC * NS

EP = 100352
EP2 = 114688
EROWS = EP2 // 128
WROWS = EROWS // NS
SEG_PAD = 25088
OUT_PAD = 25088
EG_PAD = 80
ZCH = SEG_PAD // NS

GCH = EP // NW
GSTEP = 112
GN = GCH // GSTEP

_PREC = lax.Precision.DEFAULT

_MESH = plsc.VectorSubcoreMesh(core_axis_name="c", subcore_axis_name="s")

_SC_CP = pltpu.CompilerParams()
if "needs_layout_passes" in pltpu.CompilerParams.__dataclass_fields__:
    _SC_CP = dataclasses.replace(_SC_CP, needs_layout_passes=False)


def _memproj_body(x_ref, w_ref, ol_ref, or_ref):
    a = jnp.dot(x_ref[...].astype(jnp.bfloat16),
                w_ref[...].astype(jnp.bfloat16), precision=_PREC,
                preferred_element_type=jnp.float32)
    ol_ref[...] = a[:, :NSM]
    or_ref[...] = a[:, NSM:]


def _memproj(mem, wm):
    bm = 2000
    return pl.pallas_call(
        _memproj_body,
        grid=(MEMR // bm,),
        in_specs=[
            pl.BlockSpec((bm, ND), lambda i: (i, 0)),
            pl.BlockSpec((ND, 2 * NSM), lambda i: (0, 0)),
        ],
        out_specs=[
            pl.BlockSpec((bm, NSM), lambda i: (i, 0)),
            pl.BlockSpec((bm, NSM), lambda i: (i, 0)),
        ],
        out_shape=[
            jax.ShapeDtypeStruct((MEMR, NSM), jnp.float32),
            jax.ShapeDtypeStruct((MEMR, NSM), jnp.float32),
        ],
    )(mem, wm)


def _sc_gather(al, ar, ivi, ivj):
    @functools.partial(
        pl.kernel,
        out_type=(
            jax.ShapeDtypeStruct((EP, NSM), jnp.float32),
            jax.ShapeDtypeStruct((EP, NSM), jnp.float32),
        ),
        mesh=_MESH,
        scratch_types=[
            pltpu.VMEM((GSTEP,), jnp.int32),
            pltpu.VMEM((GSTEP,), jnp.int32),
            pltpu.VMEM((GSTEP, NSM), jnp.float32),
            pltpu.VMEM((GSTEP, NSM), jnp.float32),
            pltpu.SemaphoreType.DMA,
            pltpu.SemaphoreType.DMA,
        ],
    )
    def k(al_hbm, ar_hbm, ivi_hbm, ivj_hbm, gl_hbm, gr_hbm,
          ii_v, ij_v, ri_v, rj_v, semi, semj):
        wid = lax.axis_index("s") * NC + lax.axis_index("c")
        base = wid * GCH

        @pl.loop(0, GN)
        def _(j):
            off = base + j * GSTEP
            pltpu.sync_copy(ivi_hbm.at[pl.ds(off, GSTEP)], ii_v)
            pltpu.sync_copy(ivj_hbm.at[pl.ds(off, GSTEP)], ij_v)
            ci = pltpu.async_copy(al_hbm.at[ii_v], ri_v, semi)
            cj = pltpu.async_copy(ar_hbm.at[ij_v], rj_v, semj)
            ci.wait()
            cj.wait()
            pltpu.sync_copy(ri_v, gl_hbm.at[pl.ds(off, GSTEP)])
            pltpu.sync_copy(rj_v, gr_hbm.at[pl.ds(off, GSTEP)])

    return k(al, ar, ivi, ivj)


def _edge_body(rel_ref, gl_ref, gr_ref, eg_ref, na_ref, ql_ref, qr_ref,
               wrel_ref, wc_ref, bc_ref, att_ref, gmax_ref):
    i = pl.program_id(0)
    r2 = jnp.dot(rel_ref[...].astype(jnp.bfloat16),
                 wrel_ref[...].astype(jnp.bfloat16), precision=_PREC,
                 preferred_element_type=jnp.float32)
    eg = eg_ref[0, 0, :]
    onehot = (eg[:, None] ==
              lax.broadcasted_iota(jnp.int32, (1, NB), 1)).astype(jnp.float32)
    qlg = jnp.dot(onehot, ql_ref[...], precision=_PREC,
                  preferred_element_type=jnp.float32)
    qrg = jnp.dot(onehot, qr_ref[...], precision=_PREC,
                  preferred_element_type=jnp.float32)
    left = r2[:, :NSM] + gl_ref[...] + qlg
    right = r2[:, NSM:] + gr_ref[...] + qrg
    lh = jnp.where(left >= 0, left, 0.01 * left)
    rh = jnp.where(right >= 0, right, 0.01 * right)
    ch = jnp.dot(rh.astype(jnp.bfloat16), wc_ref[...].astype(jnp.bfloat16),
                 precision=_PREC,
                 preferred_element_type=jnp.float32) + bc_ref[...]
    logits = jnp.sum(lh * ch, axis=1)
    att = logits * na_ref[0, 0, :]
    att_ref[0, 0, :] = att
    m = jnp.max(att)

    @pl.when(i == 0)
    def _():
        gmax_ref[...] = jnp.full((128,), m, jnp.float32)

    @pl.when(i > 0)
    def _():
        gmax_ref[...] = jnp.maximum(gmax_ref[...], m)


def _edge_stage(rel, gl, gr, eg3, na3, ql, qr, wrel, wc, bc2):
    be = 2000
    ng = E // be
    return pl.pallas_call(
        _edge_body,
        grid=(ng,),
        in_specs=[
            pl.BlockSpec((be, ND), lambda i: (i, 0)),
            pl.BlockSpec((be, NSM), lambda i: (i, 0)),
            pl.BlockSpec((be, NSM), lambda i: (i, 0)),
            pl.BlockSpec((1, 1, be), lambda i: (i, 0, 0)),
            pl.BlockSpec((1, 1, be), lambda i: (i, 0, 0)),
            pl.BlockSpec((NB, NSM), lambda i: (0, 0)),
            pl.BlockSpec((NB, NSM), lambda i: (0, 0)),
            pl.BlockSpec((ND, 2 * NSM), lambda i: (0, 0)),
            pl.BlockSpec((NSM, NSM), lambda i: (0, 0)),
            pl.BlockSpec((1, NSM), lambda i: (0, 0)),
        ],
        out_specs=[
            pl.BlockSpec((1, 1, be), lambda i: (i, 0, 0)),
            pl.BlockSpec((128,), lambda i: (0,)),
        ],
        out_shape=[
            jax.ShapeDtypeStruct((ng, 1, be), jnp.float32),
            jax.ShapeDtypeStruct((128,), jnp.float32),
        ],
    )(rel, gl, gr, eg3, na3, ql, qr, wrel, wc, bc2)


def _sc_segment(att2, seg2, eg2, dst2, gmax):
    @functools.partial(
        pl.kernel,
        out_type=jax.ShapeDtypeStruct((OUT_PAD,), jnp.float32),
        mesh=_MESH,
        compiler_params=_SC_CP,
        scratch_types=[
            pltpu.VMEM_SHARED((SEG_PAD,), jnp.float32),
            pltpu.VMEM_SHARED((EG_PAD,), jnp.float32),
            pltpu.VMEM_SHARED((OUT_PAD,), jnp.float32),
            pltpu.VMEM((ZCH,), jnp.float32),
            pltpu.VMEM((WROWS, 128), jnp.float32),
            pltpu.VMEM((WROWS, 128), jnp.float32),
            pltpu.VMEM((WROWS, 128), jnp.float32),
            pltpu.VMEM((WROWS, 128), jnp.int32),
            pltpu.VMEM((WROWS, 128), jnp.int32),
            pltpu.VMEM((WROWS, 128), jnp.int32),
            pltpu.VMEM((SEG_PAD,), jnp.float32),
            pltpu.VMEM((EG_PAD,), jnp.float32),
            pltpu.VMEM((16,), jnp.float32),
            pltpu.SemaphoreType.DMA,
        ],
    )
    def k(att_hbm, seg_hbm, eg_hbm, dst_hbm, gmax_hbm, out_hbm,
          den_sh, eg_sh, out_sh, zbuf, att_v, ex_v, soft_v,
          segi_v, egi_v, dst_v, den_v, egv, gmax_v, ssem):
        core = lax.axis_index("c")

        def scatter_add_rows(val_v, idx_v, acc_sh):
            @pl.loop(0, WROWS // 7)
            def _(g):
                for u in range(7):
                    r = g * 7 + u
                    pltpu.async_copy(val_v.at[r], acc_sh.at[idx_v.at[r]],
                                     ssem, add=True)
                for u in range(7):
                    r = g * 7 + u
                    pltpu.make_async_copy(val_v.at[r],
                                          acc_sh.at[idx_v.at[r]],
                                          ssem).wait()

        @pl.when(core == 0)
        def _():
            w = lax.axis_index("s")
            base = w * WROWS

            @pl.loop(0, ZCH, step=16)
            def _(i):
                zbuf[pl.ds(i, 16)] = jnp.zeros((16,), jnp.float32)

            pltpu.sync_copy(zbuf, den_sh.at[pl.ds(w * ZCH, ZCH)])
            pltpu.sync_copy(zbuf, out_sh.at[pl.ds(w * ZCH, ZCH)])

            @pl.when(w == 0)
            def _():
                pltpu.sync_copy(zbuf.at[pl.ds(0, EG_PAD)], eg_sh)

            pltpu.sync_copy(att_hbm.at[pl.ds(base, WROWS)], att_v)
            pltpu.sync_copy(seg_hbm.at[pl.ds(base, WROWS)], segi_v)
            pltpu.sync_copy(gmax_hbm.at[pl.ds(0, 16)], gmax_v)
            plsc.subcore_barrier()

            gm = gmax_v[...]

            @pl.loop(0, WROWS)
            def _(r):
                for c in range(8):
                    sl = pl.ds(c * 16, 16)
                    ex_v[r, sl] = jnp.exp(att_v[r, sl] - gm)

            scatter_add_rows(ex_v, segi_v, den_sh)
            plsc.subcore_barrier()

            pltpu.sync_copy(den_sh, den_v)
            pltpu.sync_copy(eg_hbm.at[pl.ds(base, WROWS)], egi_v)

            @pl.loop(0, WROWS)
            def _(r):
                for c in range(8):
                    sl = pl.ds(c * 16, 16)
                    d16 = plsc.load_gather(den_v, [segi_v[r, sl]])
                    soft_v[r, sl] = ex_v[r, sl] / d16

            scatter_add_rows(soft_v, egi_v, eg_sh)
            plsc.subcore_barrier()

            pltpu.sync_copy(eg_sh, egv)
            pltpu.sync_copy(dst_hbm.at[pl.ds(base, WROWS)], dst_v)

            @pl.loop(0, WROWS)
            def _(r):
                for c in range(8):
                    sl = pl.ds(c * 16, 16)
                    e16 = plsc.load_gather(egv, [egi_v[r, sl]])
                    ex_v[r, sl] = soft_v[r, sl] / e16

            scatter_add_rows(ex_v, dst_v, out_sh)
            plsc.subcore_barrier()

            pltpu.sync_copy(out_sh.at[pl.ds(w * ZCH, ZCH)], zbuf)
            pltpu.sync_copy(zbuf, out_hbm.at[pl.ds(w * ZCH, ZCH)])

    return k(att2, seg2, eg2, dst2, gmax)


def kernel(node_attention, memorized_embedding, rel_emb, query_src_emb,
           query_rel_emb, query_time_emb, eg_idx, idx_vi, idx_vj, seg_src,
           dst_ids, W_proj, b_proj, W_left, b_left, W_right, b_right,
           W_center, b_center):
    f32 = jnp.float32
    wl = [W_left[k * NSM:(k + 1) * NSM] for k in range(5)]
    wr = [W_right[k * NSM:(k + 1) * NSM] for k in range(5)]

    wm = jnp.concatenate([W_proj @ wl[0], W_proj @ wr[0]], axis=1)
    wrel = jnp.concatenate([W_proj @ wl[1], W_proj @ wr[1]], axis=1)

    q_src = query_src_emb @ W_proj + b_proj
    q_rel = query_rel_emb @ W_proj + b_proj
    q_time = query_time_emb @ W_proj + b_proj
    ql = (q_src @ wl[2] + q_rel @ wl[3] + q_time @ wl[4]
          + b_proj @ (wl[0] + wl[1]) + b_left)
    qr = (q_src @ wr[2] + q_rel @ wr[3] + q_time @ wr[4]
          + b_proj @ (wr[0] + wr[1]) + b_right)

    al, ar = _memproj(memorized_embedding, wm)
    pad = EP - E
    ivi = jnp.concatenate([idx_vi, jnp.zeros((pad,), jnp.int32)])
    ivj = jnp.concatenate([idx_vj, jnp.zeros((pad,), jnp.int32)])
    gl, gr = _sc_gather(al, ar, ivi, ivj)

    be = 2000
    eg3 = eg_idx.reshape(E // be, 1, be)
    na3 = node_attention.reshape(E // be, 1, be)
    att3, gmax = _edge_stage(rel_emb, gl, gr, eg3, na3, ql, qr, wrel,
                             W_center, b_center.reshape(1, NSM))

    pad2 = EP2 - E
    att2 = jnp.concatenate([att3.reshape(E), jnp.zeros((pad2,), f32)])
    att2 = att2.reshape(EROWS, 128)
    seg2 = jnp.concatenate(
        [seg_src, jnp.full((pad2,), NSEG, jnp.int32)]).reshape(EROWS, 128)
    eg2 = jnp.concatenate(
        [eg_idx, jnp.full((pad2,), NB, jnp.int32)]).reshape(EROWS, 128)
    dst2 = jnp.concatenate(
        [dst_ids, jnp.full((pad2,), NTGT, jnp.int32)]).reshape(EROWS, 128)

    out_p = _sc_segment(att2, seg2, eg2, dst2, gmax)
    return out_p[:NTGT]


# speedup vs baseline: 1.0365x; 1.0365x over previous
import dataclasses
import functools

import jax
import jax.numpy as jnp
from jax import lax
from jax.experimental import pallas as pl
from jax.experimental.pallas import tpu as pltpu
from jax.experimental.pallas import tpu_sc as plsc

E = 100000
ND = 512
NSM = 128
NB = 64
NSEG = 25000
NTGT = 25000
MEMR = 40000

NC = 2
NS = 16
NW = NC * NS

EP = 100352
EP2 = 114688
EROWS = EP2 // 128
WROWS = EROWS // NS
SEG_PAD = 25088
OUT_PAD = 25088
EG_PAD = 80
ZCH = SEG_PAD // NS

GCH = EP // NW
GSTEP = 112
GN = GCH // GSTEP

_PREC = lax.Precision.DEFAULT

_MESH = plsc.VectorSubcoreMesh(core_axis_name="c", subcore_axis_name="s")

_SC_CP = pltpu.CompilerParams()
if "needs_layout_passes" in pltpu.CompilerParams.__dataclass_fields__:
    _SC_CP = dataclasses.replace(_SC_CP, needs_layout_passes=False)


def _memproj_body(x_ref, w_ref, ol_ref, or_ref):
    a = jnp.dot(x_ref[...], w_ref[...], precision=_PREC,
                preferred_element_type=jnp.float32)
    ol_ref[...] = a[:, :NSM]
    or_ref[...] = a[:, NSM:]


def _memproj(mem, wm):
    bm = 2000
    return pl.pallas_call(
        _memproj_body,
        grid=(MEMR // bm,),
        in_specs=[
            pl.BlockSpec((bm, ND), lambda i: (i, 0)),
            pl.BlockSpec((ND, 2 * NSM), lambda i: (0, 0)),
        ],
        out_specs=[
            pl.BlockSpec((bm, NSM), lambda i: (i, 0)),
            pl.BlockSpec((bm, NSM), lambda i: (i, 0)),
        ],
        out_shape=[
            jax.ShapeDtypeStruct((MEMR, NSM), jnp.float32),
            jax.ShapeDtypeStruct((MEMR, NSM), jnp.float32),
        ],
    )(mem, wm)


def _sc_gather(al, ar, ivi, ivj):
    @functools.partial(
        pl.kernel,
        out_type=(
            jax.ShapeDtypeStruct((EP, NSM), jnp.float32),
            jax.ShapeDtypeStruct((EP, NSM), jnp.float32),
        ),
        mesh=_MESH,
        scratch_types=[
            pltpu.VMEM((GSTEP,), jnp.int32),
            pltpu.VMEM((GSTEP,), jnp.int32),
            pltpu.VMEM((GSTEP, NSM), jnp.float32),
            pltpu.VMEM((GSTEP, NSM), jnp.float32),
            pltpu.SemaphoreType.DMA,
            pltpu.SemaphoreType.DMA,
        ],
    )
    def k(al_hbm, ar_hbm, ivi_hbm, ivj_hbm, gl_hbm, gr_hbm,
          ii_v, ij_v, ri_v, rj_v, semi, semj):
        wid = lax.axis_index("s") * NC + lax.axis_index("c")
        base = wid * GCH

        @pl.loop(0, GN)
        def _(j):
            off = base + j * GSTEP
            pltpu.sync_copy(ivi_hbm.at[pl.ds(off, GSTEP)], ii_v)
            pltpu.sync_copy(ivj_hbm.at[pl.ds(off, GSTEP)], ij_v)
            ci = pltpu.async_copy(al_hbm.at[ii_v], ri_v, semi)
            cj = pltpu.async_copy(ar_hbm.at[ij_v], rj_v, semj)
            ci.wait()
            cj.wait()
            pltpu.sync_copy(ri_v, gl_hbm.at[pl.ds(off, GSTEP)])
            pltpu.sync_copy(rj_v, gr_hbm.at[pl.ds(off, GSTEP)])

    return k(al, ar, ivi, ivj)


def _edge_body(rel_ref, gl_ref, gr_ref, eg_ref, na_ref, ql_ref, qr_ref,
               wrel_ref, wc_ref, bc_ref, att_ref, gmax_ref):
    i = pl.program_id(0)
    r2 = jnp.dot(rel_ref[...], wrel_ref[...], precision=_PREC,
                 preferred_element_type=jnp.float32)
    eg = eg_ref[0, 0, :]
    onehot = (eg[:, None] ==
              lax.broadcasted_iota(jnp.int32, (1, NB), 1)).astype(jnp.float32)
    qlg = jnp.dot(onehot, ql_ref[...], precision=_PREC,
                  preferred_element_type=jnp.float32)
    qrg = jnp.dot(onehot, qr_ref[...], precision=_PREC,
                  preferred_element_type=jnp.float32)
    left = r2[:, :NSM] + gl_ref[...] + qlg
    right = r2[:, NSM:] + gr_ref[...] + qrg
    lh = jnp.where(left >= 0, left, 0.01 * left)
    rh = jnp.where(right >= 0, right, 0.01 * right)
    ch = jnp.dot(rh, wc_ref[...], precision=_PREC,
                 preferred_element_type=jnp.float32) + bc_ref[...]
    logits = jnp.sum(lh * ch, axis=1)
    att = logits * na_ref[0, 0, :]
    att_ref[0, 0, :] = att
    m = jnp.max(att)

    @pl.when(i == 0)
    def _():
        gmax_ref[...] = jnp.full((128,), m, jnp.float32)

    @pl.when(i > 0)
    def _():
        gmax_ref[...] = jnp.maximum(gmax_ref[...], m)


def _edge_stage(rel, gl, gr, eg3, na3, ql, qr, wrel, wc, bc2):
    be = 2000
    ng = E // be
    return pl.pallas_call(
        _edge_body,
        grid=(ng,),
        in_specs=[
            pl.BlockSpec((be, ND), lambda i: (i, 0)),
            pl.BlockSpec((be, NSM), lambda i: (i, 0)),
            pl.BlockSpec((be, NSM), lambda i: (i, 0)),
            pl.BlockSpec((1, 1, be), lambda i: (i, 0, 0)),
            pl.BlockSpec((1, 1, be), lambda i: (i, 0, 0)),
            pl.BlockSpec((NB, NSM), lambda i: (0, 0)),
            pl.BlockSpec((NB, NSM), lambda i: (0, 0)),
            pl.BlockSpec((ND, 2 * NSM), lambda i: (0, 0)),
            pl.BlockSpec((NSM, NSM), lambda i: (0, 0)),
            pl.BlockSpec((1, NSM), lambda i: (0, 0)),
        ],
        out_specs=[
            pl.BlockSpec((1, 1, be), lambda i: (i, 0, 0)),
            pl.BlockSpec((128,), lambda i: (0,)),
        ],
        out_shape=[
            jax.ShapeDtypeStruct((ng, 1, be), jnp.float32),
            jax.ShapeDtypeStruct((128,), jnp.float32),
        ],
    )(rel, gl, gr, eg3, na3, ql, qr, wrel, wc, bc2)


def _sc_segment(att2, seg2, eg2, dst2, gmax):
    @functools.partial(
        pl.kernel,
        out_type=jax.ShapeDtypeStruct((OUT_PAD,), jnp.float32),
        mesh=_MESH,
        compiler_params=_SC_CP,
        scratch_types=[
            pltpu.VMEM_SHARED((SEG_PAD,), jnp.float32),
            pltpu.VMEM_SHARED((EG_PAD,), jnp.float32),
            pltpu.VMEM_SHARED((OUT_PAD,), jnp.float32),
            pltpu.VMEM((ZCH,), jnp.float32),
            pltpu.VMEM((WROWS, 128), jnp.float32),
            pltpu.VMEM((WROWS, 128), jnp.float32),
            pltpu.VMEM((WROWS, 128), jnp.float32),
            pltpu.VMEM((WROWS, 128), jnp.int32),
            pltpu.VMEM((WROWS, 128), jnp.int32),
            pltpu.VMEM((WROWS, 128), jnp.int32),
            pltpu.VMEM((SEG_PAD,), jnp.float32),
            pltpu.VMEM((EG_PAD,), jnp.float32),
            pltpu.VMEM((16,), jnp.float32),
            pltpu.SemaphoreType.DMA,
        ],
    )
    def k(att_hbm, seg_hbm, eg_hbm, dst_hbm, gmax_hbm, out_hbm,
          den_sh, eg_sh, out_sh, zbuf, att_v, ex_v, soft_v,
          segi_v, egi_v, dst_v, den_v, egv, gmax_v, ssem):
        core = lax.axis_index("c")

        def scatter_add_rows(val_v, idx_v, acc_sh):
            @pl.loop(0, WROWS // 7)
            def _(g):
                for u in range(7):
                    r = g * 7 + u
                    pltpu.async_copy(val_v.at[r], acc_sh.at[idx_v.at[r]],
                                     ssem, add=True)
                for u in range(7):
                    r = g * 7 + u
                    pltpu.make_async_copy(val_v.at[r],
                                          acc_sh.at[idx_v.at[r]],
                                          ssem).wait()

        @pl.when(core == 0)
        def _():
            w = lax.axis_index("s")
            base = w * WROWS

            @pl.loop(0, ZCH, step=16)
            def _(i):
                zbuf[pl.ds(i, 16)] = jnp.zeros((16,), jnp.float32)

            pltpu.sync_copy(zbuf, den_sh.at[pl.ds(w * ZCH, ZCH)])
            pltpu.sync_copy(zbuf, out_sh.at[pl.ds(w * ZCH, ZCH)])

            @pl.when(w == 0)
            def _():
                pltpu.sync_copy(zbuf.at[pl.ds(0, EG_PAD)], eg_sh)

            pltpu.sync_copy(att_hbm.at[pl.ds(base, WROWS)], att_v)
            pltpu.sync_copy(seg_hbm.at[pl.ds(base, WROWS)], segi_v)
            pltpu.sync_copy(gmax_hbm.at[pl.ds(0, 16)], gmax_v)
            plsc.subcore_barrier()

            gm = gmax_v[...]

            @pl.loop(0, WROWS)
            def _(r):
                for c in range(8):
                    sl = pl.ds(c * 16, 16)
                    ex_v[r, sl] = jnp.exp(att_v[r, sl] - gm)

            scatter_add_rows(ex_v, segi_v, den_sh)
            plsc.subcore_barrier()

            pltpu.sync_copy(den_sh, den_v)
            pltpu.sync_copy(eg_hbm.at[pl.ds(base, WROWS)], egi_v)

            @pl.loop(0, WROWS)
            def _(r):
                for c in range(8):
                    sl = pl.ds(c * 16, 16)
                    d16 = plsc.load_gather(den_v, [segi_v[r, sl]])
                    soft_v[r, sl] = ex_v[r, sl] / d16

            scatter_add_rows(soft_v, egi_v, eg_sh)
            plsc.subcore_barrier()

            pltpu.sync_copy(eg_sh, egv)
            pltpu.sync_copy(dst_hbm.at[pl.ds(base, WROWS)], dst_v)

            @pl.loop(0, WROWS)
            def _(r):
                for c in range(8):
                    sl = pl.ds(c * 16, 16)
                    e16 = plsc.load_gather(egv, [egi_v[r, sl]])
                    ex_v[r, sl] = soft_v[r, sl] / e16

            scatter_add_rows(ex_v, dst_v, out_sh)
            plsc.subcore_barrier()

            pltpu.sync_copy(out_sh.at[pl.ds(w * ZCH, ZCH)], zbuf)
            pltpu.sync_copy(zbuf, out_hbm.at[pl.ds(w * ZCH, ZCH)])

    return k(att2, seg2, eg2, dst2, gmax)


def kernel(node_attention, memorized_embedding, rel_emb, query_src_emb,
           query_rel_emb, query_time_emb, eg_idx, idx_vi, idx_vj, seg_src,
           dst_ids, W_proj, b_proj, W_left, b_left, W_right, b_right,
           W_center, b_center):
    f32 = jnp.float32
    wl = [W_left[k * NSM:(k + 1) * NSM] for k in range(5)]
    wr = [W_right[k * NSM:(k + 1) * NSM] for k in range(5)]

    wm = jnp.concatenate([W_proj @ wl[0], W_proj @ wr[0]], axis=1)
    wrel = jnp.concatenate([W_proj @ wl[1], W_proj @ wr[1]], axis=1)

    q_src = query_src_emb @ W_proj + b_proj
    q_rel = query_rel_emb @ W_proj + b_proj
    q_time = query_time_emb @ W_proj + b_proj
    ql = (q_src @ wl[2] + q_rel @ wl[3] + q_time @ wl[4]
          + b_proj @ (wl[0] + wl[1]) + b_left)
    qr = (q_src @ wr[2] + q_rel @ wr[3] + q_time @ wr[4]
          + b_proj @ (wr[0] + wr[1]) + b_right)

    al, ar = _memproj(memorized_embedding, wm)
    pad = EP - E
    ivi = jnp.concatenate([idx_vi, jnp.zeros((pad,), jnp.int32)])
    ivj = jnp.concatenate([idx_vj, jnp.zeros((pad,), jnp.int32)])
    gl, gr = _sc_gather(al, ar, ivi, ivj)

    be = 2000
    eg3 = eg_idx.reshape(E // be, 1, be)
    na3 = node_attention.reshape(E // be, 1, be)
    att3, gmax = _edge_stage(rel_emb, gl, gr, eg3, na3, ql, qr, wrel,
                             W_center, b_center.reshape(1, NSM))

    pad2 = EP2 - E
    att2 = jnp.concatenate([att3.reshape(E), jnp.zeros((pad2,), f32)])
    att2 = att2.reshape(EROWS, 128)
    seg2 = jnp.concatenate(
        [seg_src, jnp.full((pad2,), NSEG, jnp.int32)]).reshape(EROWS, 128)
    eg2 = jnp.concatenate(
        [eg_idx, jnp.full((pad2,), NB, jnp.int32)]).reshape(EROWS, 128)
    dst2 = jnp.concatenate(
        [dst_ids, jnp.full((pad2,), NTGT, jnp.int32)]).reshape(EROWS, 128)

    out_p = _sc_segment(att2, seg2, eg2, dst2, gmax)
    return out_p[:NTGT]


# speedup vs baseline: 1.0930x; 1.0545x over previous
import dataclasses
import functools

import jax
import jax.numpy as jnp
from jax import lax
from jax.experimental import pallas as pl
from jax.experimental.pallas import tpu as pltpu
from jax.experimental.pallas import tpu_sc as plsc

E = 100000
ND = 512
NSM = 128
NB = 64
NSEG = 25000
NTGT = 25000
MEMR = 40000

NC = 2
NS = 16
NW = NC * NS

EP = 100352
EP2 = 114688
EROWS = EP2 // 128
WROWS = EROWS // NS
SEG_PAD = 25088
OUT_PAD = 25088
EG_PAD = 80
ZCH = SEG_PAD // NS

GCH = EP // NW
GSTEP = 112
GN = GCH // GSTEP

_PREC = lax.Precision.DEFAULT

_MESH = plsc.VectorSubcoreMesh(core_axis_name="c", subcore_axis_name="s")

_SC_CP = pltpu.CompilerParams()
if "needs_layout_passes" in pltpu.CompilerParams.__dataclass_fields__:
    _SC_CP = dataclasses.replace(_SC_CP, needs_layout_passes=False)


def _memproj_body(x_ref, w_ref, ol_ref, or_ref):
    a = jnp.dot(x_ref[...], w_ref[...], precision=_PREC,
                preferred_element_type=jnp.float32)
    ol_ref[...] = a[:, :NSM]
    or_ref[...] = a[:, NSM:]


def _memproj(mem, wm):
    bm = 2000
    return pl.pallas_call(
        _memproj_body,
        grid=(MEMR // bm,),
        in_specs=[
            pl.BlockSpec((bm, ND), lambda i: (i, 0)),
            pl.BlockSpec((ND, 2 * NSM), lambda i: (0, 0)),
        ],
        out_specs=[
            pl.BlockSpec((bm, NSM), lambda i: (i, 0)),
            pl.BlockSpec((bm, NSM), lambda i: (i, 0)),
        ],
        out_shape=[
            jax.ShapeDtypeStruct((MEMR, NSM), jnp.float32),
            jax.ShapeDtypeStruct((MEMR, NSM), jnp.float32),
        ],
    )(mem, wm)


def _sc_gather(al, ar, ivi, ivj, nrows, gstep):
    gch = nrows // NW
    gn = gch // gstep

    @functools.partial(
        pl.kernel,
        out_type=(
            jax.ShapeDtypeStruct((nrows, NSM), jnp.float32),
            jax.ShapeDtypeStruct((nrows, NSM), jnp.float32),
        ),
        mesh=_MESH,
        scratch_types=[
            pltpu.VMEM((gstep,), jnp.int32),
            pltpu.VMEM((gstep,), jnp.int32),
            pltpu.VMEM((gstep, NSM), jnp.float32),
            pltpu.VMEM((gstep, NSM), jnp.float32),
            pltpu.SemaphoreType.DMA,
            pltpu.SemaphoreType.DMA,
        ],
    )
    def k(al_hbm, ar_hbm, ivi_hbm, ivj_hbm, gl_hbm, gr_hbm,
          ii_v, ij_v, ri_v, rj_v, semi, semj):
        wid = lax.axis_index("s") * NC + lax.axis_index("c")
        base = wid * gch

        @pl.loop(0, gn)
        def _(j):
            off = base + j * gstep
            pltpu.sync_copy(ivi_hbm.at[pl.ds(off, gstep)], ii_v)
            pltpu.sync_copy(ivj_hbm.at[pl.ds(off, gstep)], ij_v)
            ci = pltpu.async_copy(al_hbm.at[ii_v], ri_v, semi)
            cj = pltpu.async_copy(ar_hbm.at[ij_v], rj_v, semj)
            ci.wait()
            cj.wait()
            pltpu.sync_copy(ri_v, gl_hbm.at[pl.ds(off, gstep)])
            pltpu.sync_copy(rj_v, gr_hbm.at[pl.ds(off, gstep)])

    return k(al, ar, ivi, ivj)


def _edge_body(rel_ref, gl_ref, gr_ref, eg_ref, na_ref, ql_ref, qr_ref,
               wrel_ref, wc_ref, bc_ref, att_ref, gmax_ref):
    i = pl.program_id(0)
    r2 = jnp.dot(rel_ref[...], wrel_ref[...], precision=_PREC,
                 preferred_element_type=jnp.float32)
    eg = eg_ref[0, 0, :]
    onehot = (eg[:, None] ==
              lax.broadcasted_iota(jnp.int32, (1, NB), 1)).astype(jnp.float32)
    qlg = jnp.dot(onehot, ql_ref[...], precision=_PREC,
                  preferred_element_type=jnp.float32)
    qrg = jnp.dot(onehot, qr_ref[...], precision=_PREC,
                  preferred_element_type=jnp.float32)
    left = r2[:, :NSM] + gl_ref[...] + qlg
    right = r2[:, NSM:] + gr_ref[...] + qrg
    lh = jnp.where(left >= 0, left, 0.01 * left)
    rh = jnp.where(right >= 0, right, 0.01 * right)
    ch = jnp.dot(rh, wc_ref[...], precision=_PREC,
                 preferred_element_type=jnp.float32) + bc_ref[...]
    logits = jnp.sum(lh * ch, axis=1)
    att = logits * na_ref[0, 0, :]
    att_ref[0, 0, :] = att
    m = jnp.max(att)

    @pl.when(i == 0)
    def _():
        gmax_ref[...] = jnp.full((128,), m, jnp.float32)

    @pl.when(i > 0)
    def _():
        gmax_ref[...] = jnp.maximum(gmax_ref[...], m)


def _edge_stage(rel, gl, gr, eg3, na3, ql, qr, wrel, wc, bc2, be, ng, offb):
    return pl.pallas_call(
        _edge_body,
        grid=(ng,),
        in_specs=[
            pl.BlockSpec((be, ND), lambda i: (i + offb, 0)),
            pl.BlockSpec((be, NSM), lambda i: (i, 0)),
            pl.BlockSpec((be, NSM), lambda i: (i, 0)),
            pl.BlockSpec((1, 1, be), lambda i: (i, 0, 0)),
            pl.BlockSpec((1, 1, be), lambda i: (i, 0, 0)),
            pl.BlockSpec((NB, NSM), lambda i: (0, 0)),
            pl.BlockSpec((NB, NSM), lambda i: (0, 0)),
            pl.BlockSpec((ND, 2 * NSM), lambda i: (0, 0)),
            pl.BlockSpec((NSM, NSM), lambda i: (0, 0)),
            pl.BlockSpec((1, NSM), lambda i: (0, 0)),
        ],
        out_specs=[
            pl.BlockSpec((1, 1, be), lambda i: (i, 0, 0)),
            pl.BlockSpec((128,), lambda i: (0,)),
        ],
        out_shape=[
            jax.ShapeDtypeStruct((ng, 1, be), jnp.float32),
            jax.ShapeDtypeStruct((128,), jnp.float32),
        ],
    )(rel, gl, gr, eg3, na3, ql, qr, wrel, wc, bc2)


def _sc_segment(att2, seg2, eg2, dst2, gmax):
    @functools.partial(
        pl.kernel,
        out_type=jax.ShapeDtypeStruct((OUT_PAD,), jnp.float32),
        mesh=_MESH,
        compiler_params=_SC_CP,
        scratch_types=[
            pltpu.VMEM_SHARED((SEG_PAD,), jnp.float32),
            pltpu.VMEM_SHARED((EG_PAD,), jnp.float32),
            pltpu.VMEM_SHARED((OUT_PAD,), jnp.float32),
            pltpu.VMEM((ZCH,), jnp.float32),
            pltpu.VMEM((WROWS, 128), jnp.float32),
            pltpu.VMEM((WROWS, 128), jnp.float32),
            pltpu.VMEM((WROWS, 128), jnp.float32),
            pltpu.VMEM((WROWS, 128), jnp.int32),
            pltpu.VMEM((WROWS, 128), jnp.int32),
            pltpu.VMEM((WROWS, 128), jnp.int32),
            pltpu.VMEM((SEG_PAD,), jnp.float32),
            pltpu.VMEM((EG_PAD,), jnp.float32),
            pltpu.VMEM((16,), jnp.float32),
            pltpu.SemaphoreType.DMA,
        ],
    )
    def k(att_hbm, seg_hbm, eg_hbm, dst_hbm, gmax_hbm, out_hbm,
          den_sh, eg_sh, out_sh, zbuf, att_v, ex_v, soft_v,
          segi_v, egi_v, dst_v, den_v, egv, gmax_v, ssem):
        core = lax.axis_index("c")

        def scatter_add_rows(val_v, idx_v, acc_sh):
            @pl.loop(0, WROWS // 7)
            def _(g):
                for u in range(7):
                    r = g * 7 + u
                    pltpu.async_copy(val_v.at[r], acc_sh.at[idx_v.at[r]],
                                     ssem, add=True)
                for u in range(7):
                    r = g * 7 + u
                    pltpu.make_async_copy(val_v.at[r],
                                          acc_sh.at[idx_v.at[r]],
                                          ssem).wait()

        @pl.when(core == 0)
        def _():
            w = lax.axis_index("s")
            base = w * WROWS

            @pl.loop(0, ZCH, step=16)
            def _(i):
                zbuf[pl.ds(i, 16)] = jnp.zeros((16,), jnp.float32)

            pltpu.sync_copy(zbuf, den_sh.at[pl.ds(w * ZCH, ZCH)])
            pltpu.sync_copy(zbuf, out_sh.at[pl.ds(w * ZCH, ZCH)])

            @pl.when(w == 0)
            def _():
                pltpu.sync_copy(zbuf.at[pl.ds(0, EG_PAD)], eg_sh)

            pltpu.sync_copy(att_hbm.at[pl.ds(base, WROWS)], att_v)
            pltpu.sync_copy(seg_hbm.at[pl.ds(base, WROWS)], segi_v)
            pltpu.sync_copy(gmax_hbm.at[pl.ds(0, 16)], gmax_v)
            plsc.subcore_barrier()

            gm = gmax_v[...]

            @pl.loop(0, WROWS)
            def _(r):
                for c in range(8):
                    sl = pl.ds(c * 16, 16)
                    ex_v[r, sl] = jnp.exp(att_v[r, sl] - gm)

            scatter_add_rows(ex_v, segi_v, den_sh)
            plsc.subcore_barrier()

            pltpu.sync_copy(den_sh, den_v)
            pltpu.sync_copy(eg_hbm.at[pl.ds(base, WROWS)], egi_v)

            @pl.loop(0, WROWS)
            def _(r):
                for c in range(8):
                    sl = pl.ds(c * 16, 16)
                    d16 = plsc.load_gather(den_v, [segi_v[r, sl]])
                    soft_v[r, sl] = ex_v[r, sl] / d16

            scatter_add_rows(soft_v, egi_v, eg_sh)
            plsc.subcore_barrier()

            pltpu.sync_copy(eg_sh, egv)
            pltpu.sync_copy(dst_hbm.at[pl.ds(base, WROWS)], dst_v)

            @pl.loop(0, WROWS)
            def _(r):
                for c in range(8):
                    sl = pl.ds(c * 16, 16)
                    e16 = plsc.load_gather(egv, [egi_v[r, sl]])
                    ex_v[r, sl] = soft_v[r, sl] / e16

            scatter_add_rows(ex_v, dst_v, out_sh)
            plsc.subcore_barrier()

            pltpu.sync_copy(out_sh.at[pl.ds(w * ZCH, ZCH)], zbuf)
            pltpu.sync_copy(zbuf, out_hbm.at[pl.ds(w * ZCH, ZCH)])

    return k(att2, seg2, eg2, dst2, gmax)


def kernel(node_attention, memorized_embedding, rel_emb, query_src_emb,
           query_rel_emb, query_time_emb, eg_idx, idx_vi, idx_vj, seg_src,
           dst_ids, W_proj, b_proj, W_left, b_left, W_right, b_right,
           W_center, b_center):
    f32 = jnp.float32
    wl = [W_left[k * NSM:(k + 1) * NSM] for k in range(5)]
    wr = [W_right[k * NSM:(k + 1) * NSM] for k in range(5)]

    wm = jnp.concatenate([W_proj @ wl[0], W_proj @ wr[0]], axis=1)
    wrel = jnp.concatenate([W_proj @ wl[1], W_proj @ wr[1]], axis=1)

    q_src = query_src_emb @ W_proj + b_proj
    q_rel = query_rel_emb @ W_proj + b_proj
    q_time = query_time_emb @ W_proj + b_proj
    ql = (q_src @ wl[2] + q_rel @ wl[3] + q_time @ wl[4]
          + b_proj @ (wl[0] + wl[1]) + b_left)
    qr = (q_src @ wr[2] + q_rel @ wr[3] + q_time @ wr[4]
          + b_proj @ (wr[0] + wr[1]) + b_right)

    al, ar = _memproj(memorized_embedding, wm)
    pad = EP - E
    ivi = jnp.concatenate([idx_vi, jnp.zeros((pad,), jnp.int32)])
    ivj = jnp.concatenate([idx_vj, jnp.zeros((pad,), jnp.int32)])

    CUT = 51200
    bc2 = b_center.reshape(1, NSM)
    gl0, gr0 = _sc_gather(al, ar, ivi[:CUT], ivj[:CUT], CUT, 80)
    gl1, gr1 = _sc_gather(al, ar, ivi[CUT:], ivj[CUT:], EP - CUT, 96)

    eg0 = eg_idx[:CUT].reshape(32, 1, 1600)
    na0 = node_attention[:CUT].reshape(32, 1, 1600)
    att0, gmax0 = _edge_stage(rel_emb, gl0, gr0, eg0, na0, ql, qr, wrel,
                              W_center, bc2, 1600, 32, 0)
    eg1 = eg_idx[CUT:].reshape(61, 1, 800)
    na1 = node_attention[CUT:].reshape(61, 1, 800)
    att1, gmax1 = _edge_stage(rel_emb, gl1, gr1, eg1,
                              na1, ql, qr, wrel, W_center, bc2, 800, 61, 64)
    gmax = jnp.maximum(gmax0, gmax1)

    pad2 = EP2 - E
    att2 = jnp.concatenate(
        [att0.reshape(CUT), att1.reshape(E - CUT), jnp.zeros((pad2,), f32)])
    att2 = att2.reshape(EROWS, 128)
    seg2 = jnp.concatenate(
        [seg_src, jnp.full((pad2,), NSEG, jnp.int32)]).reshape(EROWS, 128)
    eg2 = jnp.concatenate(
        [eg_idx, jnp.full((pad2,), NB, jnp.int32)]).reshape(EROWS, 128)
    dst2 = jnp.concatenate(
        [dst_ids, jnp.full((pad2,), NTGT, jnp.int32)]).reshape(EROWS, 128)

    out_p = _sc_segment(att2, seg2, eg2, dst2, gmax)
    return out_p[:NTGT]


# speedup vs baseline: 1.1537x; 1.0556x over previous
import dataclasses
import functools

import jax
import jax.numpy as jnp
from jax import lax
from jax.experimental import pallas as pl
from jax.experimental.pallas import tpu as pltpu
from jax.experimental.pallas import tpu_sc as plsc

E = 100000
ND = 512
NSM = 128
NB = 64
NSEG = 25000
NTGT = 25000
MEMR = 40000

NC = 2
NS = 16
NW = NC * NS

EP = 100352
EP2 = 114688
EROWS = EP2 // 128
WROWS = EROWS // NS
SEG_PAD = 25088
OUT_PAD = 25088
EG_PAD = 80
ZCH = SEG_PAD // NS

GCH = EP // NW
GSTEP = 112
GN = GCH // GSTEP

_PREC = lax.Precision.DEFAULT

_MESH = plsc.VectorSubcoreMesh(core_axis_name="c", subcore_axis_name="s")

_SC_CP = pltpu.CompilerParams()
if "needs_layout_passes" in pltpu.CompilerParams.__dataclass_fields__:
    _SC_CP = dataclasses.replace(_SC_CP, needs_layout_passes=False)


def _memproj_body(x_ref, w_ref, ol_ref, or_ref):
    a = jnp.dot(x_ref[...], w_ref[...], precision=_PREC,
                preferred_element_type=jnp.float32)
    ol_ref[...] = a[:, :NSM]
    or_ref[...] = a[:, NSM:]


def _memproj(mem, wm):
    bm = 2000
    return pl.pallas_call(
        _memproj_body,
        grid=(MEMR // bm,),
        in_specs=[
            pl.BlockSpec((bm, ND), lambda i: (i, 0)),
            pl.BlockSpec((ND, 2 * NSM), lambda i: (0, 0)),
        ],
        out_specs=[
            pl.BlockSpec((bm, NSM), lambda i: (i, 0)),
            pl.BlockSpec((bm, NSM), lambda i: (i, 0)),
        ],
        out_shape=[
            jax.ShapeDtypeStruct((MEMR, NSM), jnp.float32),
            jax.ShapeDtypeStruct((MEMR, NSM), jnp.float32),
        ],
    )(mem, wm)


def _sc_gather(al, ar, ivi, ivj, nrows, gstep):
    gch = nrows // NW
    gn = gch // gstep

    @functools.partial(
        pl.kernel,
        out_type=(
            jax.ShapeDtypeStruct((nrows, NSM), jnp.float32),
            jax.ShapeDtypeStruct((nrows, NSM), jnp.float32),
        ),
        mesh=_MESH,
        scratch_types=[
            pltpu.VMEM((gstep,), jnp.int32),
            pltpu.VMEM((gstep,), jnp.int32),
            pltpu.VMEM((gstep, NSM), jnp.float32),
            pltpu.VMEM((gstep, NSM), jnp.float32),
            pltpu.SemaphoreType.DMA,
            pltpu.SemaphoreType.DMA,
        ],
    )
    def k(al_hbm, ar_hbm, ivi_hbm, ivj_hbm, gl_hbm, gr_hbm,
          ii_v, ij_v, ri_v, rj_v, semi, semj):
        wid = lax.axis_index("s") * NC + lax.axis_index("c")
        base = wid * gch

        @pl.loop(0, gn)
        def _(j):
            off = base + j * gstep
            pltpu.sync_copy(ivi_hbm.at[pl.ds(off, gstep)], ii_v)
            pltpu.sync_copy(ivj_hbm.at[pl.ds(off, gstep)], ij_v)
            ci = pltpu.async_copy(al_hbm.at[ii_v], ri_v, semi)
            cj = pltpu.async_copy(ar_hbm.at[ij_v], rj_v, semj)
            ci.wait()
            cj.wait()
            pltpu.sync_copy(ri_v, gl_hbm.at[pl.ds(off, gstep)])
            pltpu.sync_copy(rj_v, gr_hbm.at[pl.ds(off, gstep)])

    return k(al, ar, ivi, ivj)


def _edge_body(rel_ref, gl_ref, gr_ref, eg_ref, na_ref, ql_ref, qr_ref,
               wrel_ref, wc_ref, bc_ref, att_ref):
    r2 = jnp.dot(rel_ref[...], wrel_ref[...], precision=_PREC,
                 preferred_element_type=jnp.float32)
    eg = eg_ref[0, 0, :]
    onehot = (eg[:, None] ==
              lax.broadcasted_iota(jnp.int32, (1, NB), 1)).astype(jnp.float32)
    qlg = jnp.dot(onehot, ql_ref[...], precision=_PREC,
                  preferred_element_type=jnp.float32)
    qrg = jnp.dot(onehot, qr_ref[...], precision=_PREC,
                  preferred_element_type=jnp.float32)
    left = r2[:, :NSM] + gl_ref[...] + qlg
    right = r2[:, NSM:] + gr_ref[...] + qrg
    lh = jnp.where(left >= 0, left, 0.01 * left)
    rh = jnp.where(right >= 0, right, 0.01 * right)
    ch = jnp.dot(rh, wc_ref[...], precision=_PREC,
                 preferred_element_type=jnp.float32) + bc_ref[...]
    logits = jnp.sum(lh * ch, axis=1)
    att_ref[0, 0, :] = logits * na_ref[0, 0, :]


def _edge_stage(rel, gl, gr, eg3, na3, ql, qr, wrel, wc, bc2, be, ng, offb):
    return pl.pallas_call(
        _edge_body,
        grid=(ng,),
        in_specs=[
            pl.BlockSpec((be, ND), lambda i: (i + offb, 0)),
            pl.BlockSpec((be, NSM), lambda i: (i, 0)),
            pl.BlockSpec((be, NSM), lambda i: (i, 0)),
            pl.BlockSpec((1, 1, be), lambda i: (i, 0, 0)),
            pl.BlockSpec((1, 1, be), lambda i: (i, 0, 0)),
            pl.BlockSpec((NB, NSM), lambda i: (0, 0)),
            pl.BlockSpec((NB, NSM), lambda i: (0, 0)),
            pl.BlockSpec((ND, 2 * NSM), lambda i: (0, 0)),
            pl.BlockSpec((NSM, NSM), lambda i: (0, 0)),
            pl.BlockSpec((1, NSM), lambda i: (0, 0)),
        ],
        out_specs=pl.BlockSpec((1, 1, be), lambda i: (i, 0, 0)),
        out_shape=jax.ShapeDtypeStruct((ng, 1, be), jnp.float32),
    )(rel, gl, gr, eg3, na3, ql, qr, wrel, wc, bc2)


def _sc_segment(att2, seg2, eg2, dst2):
    @functools.partial(
        pl.kernel,
        out_type=jax.ShapeDtypeStruct((OUT_PAD,), jnp.float32),
        mesh=_MESH,
        compiler_params=_SC_CP,
        scratch_types=[
            pltpu.VMEM_SHARED((SEG_PAD,), jnp.float32),
            pltpu.VMEM_SHARED((EG_PAD,), jnp.float32),
            pltpu.VMEM_SHARED((OUT_PAD,), jnp.float32),
            pltpu.VMEM_SHARED((NS, 16), jnp.float32),
            pltpu.VMEM((ZCH,), jnp.float32),
            pltpu.VMEM((WROWS, 128), jnp.float32),
            pltpu.VMEM((WROWS, 128), jnp.float32),
            pltpu.VMEM((WROWS, 128), jnp.float32),
            pltpu.VMEM((WROWS, 128), jnp.int32),
            pltpu.VMEM((WROWS, 128), jnp.int32),
            pltpu.VMEM((WROWS, 128), jnp.int32),
            pltpu.VMEM((SEG_PAD,), jnp.float32),
            pltpu.VMEM((EG_PAD,), jnp.float32),
            pltpu.VMEM((16,), jnp.float32),
            pltpu.VMEM((NS, 16), jnp.float32),
            pltpu.SemaphoreType.DMA,
        ],
    )
    def k(att_hbm, seg_hbm, eg_hbm, dst_hbm, out_hbm,
          den_sh, eg_sh, out_sh, max_sh, zbuf, att_v, ex_v, soft_v,
          segi_v, egi_v, dst_v, den_v, egv, gmax_v, slab_v, ssem):
        core = lax.axis_index("c")

        def scatter_add_rows(val_v, idx_v, acc_sh):
            @pl.loop(0, WROWS // 7)
            def _(g):
                for u in range(7):
                    r = g * 7 + u
                    pltpu.async_copy(val_v.at[r], acc_sh.at[idx_v.at[r]],
                                     ssem, add=True)
                for u in range(7):
                    r = g * 7 + u
                    pltpu.make_async_copy(val_v.at[r],
                                          acc_sh.at[idx_v.at[r]],
                                          ssem).wait()

        @pl.when(core == 0)
        def _():
            w = lax.axis_index("s")
            base = w * WROWS

            @pl.loop(0, ZCH, step=16)
            def _(i):
                zbuf[pl.ds(i, 16)] = jnp.zeros((16,), jnp.float32)

            pltpu.sync_copy(zbuf, den_sh.at[pl.ds(w * ZCH, ZCH)])
            pltpu.sync_copy(zbuf, out_sh.at[pl.ds(w * ZCH, ZCH)])

            @pl.when(w == 0)
            def _():
                pltpu.sync_copy(zbuf.at[pl.ds(0, EG_PAD)], eg_sh)

            pltpu.sync_copy(att_hbm.at[pl.ds(base, WROWS)], att_v)
            pltpu.sync_copy(seg_hbm.at[pl.ds(base, WROWS)], segi_v)

            gmax_v[...] = att_v[0, pl.ds(0, 16)]

            @pl.loop(0, WROWS)
            def _(r):
                for c in range(8):
                    gmax_v[...] = jnp.maximum(gmax_v[...],
                                              att_v[r, pl.ds(c * 16, 16)])

            mloc = lax.reduce_max(gmax_v[...], axes=(0,))
            gmax_v[...] = jnp.full((16,), mloc, jnp.float32)
            pltpu.sync_copy(gmax_v, max_sh.at[w])
            plsc.subcore_barrier()

            pltpu.sync_copy(max_sh, slab_v)
            gmax_v[...] = slab_v[0, pl.ds(0, 16)]
            for t in range(1, NS):
                gmax_v[...] = jnp.maximum(gmax_v[...], slab_v[t, pl.ds(0, 16)])
            gm = jnp.full((16,), lax.reduce_max(gmax_v[...], axes=(0,)),
                          jnp.float32)

            @pl.loop(0, WROWS)
            def _(r):
                for c in range(8):
                    sl = pl.ds(c * 16, 16)
                    ex_v[r, sl] = jnp.exp(att_v[r, sl] - gm)

            scatter_add_rows(ex_v, segi_v, den_sh)
            plsc.subcore_barrier()

            pltpu.sync_copy(den_sh, den_v)
            pltpu.sync_copy(eg_hbm.at[pl.ds(base, WROWS)], egi_v)

            @pl.loop(0, WROWS)
            def _(r):
                for c in range(8):
                    sl = pl.ds(c * 16, 16)
                    d16 = plsc.load_gather(den_v, [segi_v[r, sl]])
                    soft_v[r, sl] = ex_v[r, sl] / d16

            scatter_add_rows(soft_v, egi_v, eg_sh)
            plsc.subcore_barrier()

            pltpu.sync_copy(eg_sh, egv)
            pltpu.sync_copy(dst_hbm.at[pl.ds(base, WROWS)], dst_v)

            @pl.loop(0, WROWS)
            def _(r):
                for c in range(8):
                    sl = pl.ds(c * 16, 16)
                    e16 = plsc.load_gather(egv, [egi_v[r, sl]])
                    ex_v[r, sl] = soft_v[r, sl] / e16

            scatter_add_rows(ex_v, dst_v, out_sh)
            plsc.subcore_barrier()

            pltpu.sync_copy(out_sh.at[pl.ds(w * ZCH, ZCH)], zbuf)
            pltpu.sync_copy(zbuf, out_hbm.at[pl.ds(w * ZCH, ZCH)])

    return k(att2, seg2, eg2, dst2)


def kernel(node_attention, memorized_embedding, rel_emb, query_src_emb,
           query_rel_emb, query_time_emb, eg_idx, idx_vi, idx_vj, seg_src,
           dst_ids, W_proj, b_proj, W_left, b_left, W_right, b_right,
           W_center, b_center):
    f32 = jnp.float32
    wl = [W_left[k * NSM:(k + 1) * NSM] for k in range(5)]
    wr = [W_right[k * NSM:(k + 1) * NSM] for k in range(5)]

    wm = jnp.concatenate([W_proj @ wl[0], W_proj @ wr[0]], axis=1)
    wrel = jnp.concatenate([W_proj @ wl[1], W_proj @ wr[1]], axis=1)

    q_src = query_src_emb @ W_proj + b_proj
    q_rel = query_rel_emb @ W_proj + b_proj
    q_time = query_time_emb @ W_proj + b_proj
    ql = (q_src @ wl[2] + q_rel @ wl[3] + q_time @ wl[4]
          + b_proj @ (wl[0] + wl[1]) + b_left)
    qr = (q_src @ wr[2] + q_rel @ wr[3] + q_time @ wr[4]
          + b_proj @ (wr[0] + wr[1]) + b_right)

    al, ar = _memproj(memorized_embedding, wm)
    pad = EP - E
    ivi = jnp.concatenate([idx_vi, jnp.zeros((pad,), jnp.int32)])
    ivj = jnp.concatenate([idx_vj, jnp.zeros((pad,), jnp.int32)])

    CUT = 51200
    bc2 = b_center.reshape(1, NSM)
    gl0, gr0 = _sc_gather(al, ar, ivi[:CUT], ivj[:CUT], CUT, 80)
    gl1, gr1 = _sc_gather(al, ar, ivi[CUT:], ivj[CUT:], EP - CUT, 96)

    eg0 = eg_idx[:CUT].reshape(32, 1, 1600)
    na0 = node_attention[:CUT].reshape(32, 1, 1600)
    att0 = _edge_stage(rel_emb, gl0, gr0, eg0, na0, ql, qr, wrel,
                       W_center, bc2, 1600, 32, 0)
    eg1 = eg_idx[CUT:].reshape(61, 1, 800)
    na1 = node_attention[CUT:].reshape(61, 1, 800)
    att1 = _edge_stage(rel_emb, gl1, gr1, eg1,
                       na1, ql, qr, wrel, W_center, bc2, 800, 61, 64)

    pad2 = EP2 - E
    att2 = jnp.concatenate(
        [att0.reshape(CUT), att1.reshape(E - CUT), jnp.zeros((pad2,), f32)])
    att2 = att2.reshape(EROWS, 128)
    seg2 = jnp.concatenate(
        [seg_src, jnp.full((pad2,), NSEG, jnp.int32)]).reshape(EROWS, 128)
    eg2 = jnp.concatenate(
        [eg_idx, jnp.full((pad2,), NB, jnp.int32)]).reshape(EROWS, 128)
    dst2 = jnp.concatenate(
        [dst_ids, jnp.full((pad2,), NTGT, jnp.int32)]).reshape(EROWS, 128)

    out_p = _sc_segment(att2, seg2, eg2, dst2)
    return out_p[:NTGT]


# speedup vs baseline: 1.2076x; 1.0468x over previous
import dataclasses
import functools

import jax
import jax.numpy as jnp
from jax import lax
from jax.experimental import pallas as pl
from jax.experimental.pallas import tpu as pltpu
from jax.experimental.pallas import tpu_sc as plsc

E = 100000
ND = 512
NSM = 128
NB = 64
NSEG = 25000
NTGT = 25000
MEMR = 40000

NC = 2
NS = 16
NW = NC * NS

EP = 100352
EP2 = 114688
EROWS = EP2 // 128
WROWS = EROWS // NS
SEG_PAD = 25088
OUT_PAD = 25088
EG_PAD = 80
ZCH = SEG_PAD // NS

GCH = EP // NW
GSTEP = 112
GN = GCH // GSTEP

_PREC = lax.Precision.DEFAULT

_MESH = plsc.VectorSubcoreMesh(core_axis_name="c", subcore_axis_name="s")

_SC_CP = pltpu.CompilerParams()
if "needs_layout_passes" in pltpu.CompilerParams.__dataclass_fields__:
    _SC_CP = dataclasses.replace(_SC_CP, needs_layout_passes=False)


def _memproj_body(x_ref, w_ref, ol_ref, or_ref):
    a = jnp.dot(x_ref[...], w_ref[...], precision=_PREC,
                preferred_element_type=jnp.float32)
    ol_ref[...] = a[:, :NSM]
    or_ref[...] = a[:, NSM:]


def _memproj(mem, wm):
    bm = 2000
    return pl.pallas_call(
        _memproj_body,
        grid=(MEMR // bm,),
        in_specs=[
            pl.BlockSpec((bm, ND), lambda i: (i, 0)),
            pl.BlockSpec((ND, 2 * NSM), lambda i: (0, 0)),
        ],
        out_specs=[
            pl.BlockSpec((bm, NSM), lambda i: (i, 0)),
            pl.BlockSpec((bm, NSM), lambda i: (i, 0)),
        ],
        out_shape=[
            jax.ShapeDtypeStruct((MEMR, NSM), jnp.float32),
            jax.ShapeDtypeStruct((MEMR, NSM), jnp.float32),
        ],
    )(mem, wm)


def _sc_gather(al, ar, ivi, ivj, nrows, gstep):
    gch = nrows // NW
    gn = gch // gstep

    assert gn % 2 == 0

    @functools.partial(
        pl.kernel,
        out_type=(
            jax.ShapeDtypeStruct((nrows, NSM), jnp.float32),
            jax.ShapeDtypeStruct((nrows, NSM), jnp.float32),
        ),
        mesh=_MESH,
        scratch_types=[
            pltpu.VMEM((gstep,), jnp.int32),
            pltpu.VMEM((gstep,), jnp.int32),
            pltpu.VMEM((gstep,), jnp.int32),
            pltpu.VMEM((gstep,), jnp.int32),
            pltpu.VMEM((gstep, NSM), jnp.float32),
            pltpu.VMEM((gstep, NSM), jnp.float32),
            pltpu.VMEM((gstep, NSM), jnp.float32),
            pltpu.VMEM((gstep, NSM), jnp.float32),
            pltpu.SemaphoreType.DMA,
            pltpu.SemaphoreType.DMA,
            pltpu.SemaphoreType.DMA,
            pltpu.SemaphoreType.DMA,
        ],
    )
    def k(al_hbm, ar_hbm, ivi_hbm, ivj_hbm, gl_hbm, gr_hbm,
          ii0, ij0, ii1, ij1, ri0, rj0, ri1, rj1, si0, sj0, si1, sj1):
        wid = lax.axis_index("s") * NC + lax.axis_index("c")
        base = wid * gch

        def start(ii, ij, ri, rj, si, sj, off):
            pltpu.sync_copy(ivi_hbm.at[pl.ds(off, gstep)], ii)
            pltpu.sync_copy(ivj_hbm.at[pl.ds(off, gstep)], ij)
            pltpu.async_copy(al_hbm.at[ii], ri, si)
            pltpu.async_copy(ar_hbm.at[ij], rj, sj)

        def finish(ii, ij, ri, rj, si, sj, off):
            pltpu.make_async_copy(al_hbm.at[ii], ri, si).wait()
            pltpu.make_async_copy(ar_hbm.at[ij], rj, sj).wait()
            pltpu.sync_copy(ri, gl_hbm.at[pl.ds(off, gstep)])
            pltpu.sync_copy(rj, gr_hbm.at[pl.ds(off, gstep)])

        start(ii0, ij0, ri0, rj0, si0, sj0, base)

        @pl.loop(0, gn // 2)
        def _(g):
            j0 = base + 2 * g * gstep
            start(ii1, ij1, ri1, rj1, si1, sj1, j0 + gstep)
            finish(ii0, ij0, ri0, rj0, si0, sj0, j0)

            @pl.when(2 * g + 2 < gn)
            def _():
                start(ii0, ij0, ri0, rj0, si0, sj0, j0 + 2 * gstep)

            finish(ii1, ij1, ri1, rj1, si1, sj1, j0 + gstep)

    return k(al, ar, ivi, ivj)


def _edge_body(rel_ref, gl_ref, gr_ref, eg_ref, na_ref, ql_ref, qr_ref,
               wrel_ref, wc_ref, bc_ref, att_ref):
    r2 = jnp.dot(rel_ref[...], wrel_ref[...], precision=_PREC,
                 preferred_element_type=jnp.float32)
    eg = eg_ref[0, 0, :]
    onehot = (eg[:, None] ==
              lax.broadcasted_iota(jnp.int32, (1, NB), 1)).astype(jnp.float32)
    qlg = jnp.dot(onehot, ql_ref[...], precision=_PREC,
                  preferred_element_type=jnp.float32)
    qrg = jnp.dot(onehot, qr_ref[...], precision=_PREC,
                  preferred_element_type=jnp.float32)
    left = r2[:, :NSM] + gl_ref[...] + qlg
    right = r2[:, NSM:] + gr_ref[...] + qrg
    lh = jnp.where(left >= 0, left, 0.01 * left)
    rh = jnp.where(right >= 0, right, 0.01 * right)
    ch = jnp.dot(rh, wc_ref[...], precision=_PREC,
                 preferred_element_type=jnp.float32) + bc_ref[...]
    logits = jnp.sum(lh * ch, axis=1)
    att_ref[0, 0, :] = logits * na_ref[0, 0, :]


def _edge_stage(rel, gl, gr, eg3, na3, ql, qr, wrel, wc, bc2, be, ng, offb):
    return pl.pallas_call(
        _edge_body,
        grid=(ng,),
        in_specs=[
            pl.BlockSpec((be, ND), lambda i: (i + offb, 0)),
            pl.BlockSpec((be, NSM), lambda i: (i, 0)),
            pl.BlockSpec((be, NSM), lambda i: (i, 0)),
            pl.BlockSpec((1, 1, be), lambda i: (i, 0, 0)),
            pl.BlockSpec((1, 1, be), lambda i: (i, 0, 0)),
            pl.BlockSpec((NB, NSM), lambda i: (0, 0)),
            pl.BlockSpec((NB, NSM), lambda i: (0, 0)),
            pl.BlockSpec((ND, 2 * NSM), lambda i: (0, 0)),
            pl.BlockSpec((NSM, NSM), lambda i: (0, 0)),
            pl.BlockSpec((1, NSM), lambda i: (0, 0)),
        ],
        out_specs=pl.BlockSpec((1, 1, be), lambda i: (i, 0, 0)),
        out_shape=jax.ShapeDtypeStruct((ng, 1, be), jnp.float32),
    )(rel, gl, gr, eg3, na3, ql, qr, wrel, wc, bc2)


def _sc_segment(att2, seg2, eg2, dst2):
    @functools.partial(
        pl.kernel,
        out_type=jax.ShapeDtypeStruct((OUT_PAD,), jnp.float32),
        mesh=_MESH,
        compiler_params=_SC_CP,
        scratch_types=[
            pltpu.VMEM_SHARED((SEG_PAD,), jnp.float32),
            pltpu.VMEM_SHARED((EG_PAD,), jnp.float32),
            pltpu.VMEM_SHARED((OUT_PAD,), jnp.float32),
            pltpu.VMEM_SHARED((NS, 16), jnp.float32),
            pltpu.VMEM((ZCH,), jnp.float32),
            pltpu.VMEM((WROWS, 128), jnp.float32),
            pltpu.VMEM((WROWS, 128), jnp.float32),
            pltpu.VMEM((WROWS, 128), jnp.float32),
            pltpu.VMEM((WROWS, 128), jnp.int32),
            pltpu.VMEM((WROWS, 128), jnp.int32),
            pltpu.VMEM((WROWS, 128), jnp.int32),
            pltpu.VMEM((SEG_PAD,), jnp.float32),
            pltpu.VMEM((EG_PAD,), jnp.float32),
            pltpu.VMEM((16,), jnp.float32),
            pltpu.VMEM((NS, 16), jnp.float32),
            pltpu.SemaphoreType.DMA,
        ],
    )
    def k(att_hbm, seg_hbm, eg_hbm, dst_hbm, out_hbm,
          den_sh, eg_sh, out_sh, max_sh, zbuf, att_v, ex_v, soft_v,
          segi_v, egi_v, dst_v, den_v, egv, gmax_v, slab_v, ssem):
        core = lax.axis_index("c")

        def scatter_add_rows(val_v, idx_v, acc_sh):
            @pl.loop(0, WROWS // 7)
            def _(g):
                for u in range(7):
                    r = g * 7 + u
                    pltpu.async_copy(val_v.at[r], acc_sh.at[idx_v.at[r]],
                                     ssem, add=True)
                for u in range(7):
                    r = g * 7 + u
                    pltpu.make_async_copy(val_v.at[r],
                                          acc_sh.at[idx_v.at[r]],
                                          ssem).wait()

        @pl.when(core == 0)
        def _():
            w = lax.axis_index("s")
            base = w * WROWS

            @pl.loop(0, ZCH, step=16)
            def _(i):
                zbuf[pl.ds(i, 16)] = jnp.zeros((16,), jnp.float32)

            pltpu.sync_copy(zbuf, den_sh.at[pl.ds(w * ZCH, ZCH)])
            pltpu.sync_copy(zbuf, out_sh.at[pl.ds(w * ZCH, ZCH)])

            @pl.when(w == 0)
            def _():
                pltpu.sync_copy(zbuf.at[pl.ds(0, EG_PAD)], eg_sh)

            pltpu.sync_copy(att_hbm.at[pl.ds(base, WROWS)], att_v)
            pltpu.sync_copy(seg_hbm.at[pl.ds(base, WROWS)], segi_v)

            gmax_v[...] = att_v[0, pl.ds(0, 16)]

            @pl.loop(0, WROWS)
            def _(r):
                for c in range(8):
                    gmax_v[...] = jnp.maximum(gmax_v[...],
                                              att_v[r, pl.ds(c * 16, 16)])

            mloc = lax.reduce_max(gmax_v[...], axes=(0,))
            gmax_v[...] = jnp.full((16,), mloc, jnp.float32)
            pltpu.sync_copy(gmax_v, max_sh.at[w])
            plsc.subcore_barrier()

            pltpu.sync_copy(max_sh, slab_v)
            gmax_v[...] = slab_v[0, pl.ds(0, 16)]
            for t in range(1, NS):
                gmax_v[...] = jnp.maximum(gmax_v[...], slab_v[t, pl.ds(0, 16)])
            gm = jnp.full((16,), lax.reduce_max(gmax_v[...], axes=(0,)),
                          jnp.float32)

            @pl.loop(0, WROWS)
            def _(r):
                for c in range(8):
                    sl = pl.ds(c * 16, 16)
                    ex_v[r, sl] = jnp.exp(att_v[r, sl] - gm)

            scatter_add_rows(ex_v, segi_v, den_sh)
            plsc.subcore_barrier()

            pltpu.sync_copy(den_sh, den_v)
            pltpu.sync_copy(eg_hbm.at[pl.ds(base, WROWS)], egi_v)

            @pl.loop(0, WROWS)
            def _(r):
                for c in range(8):
                    sl = pl.ds(c * 16, 16)
                    d16 = plsc.load_gather(den_v, [segi_v[r, sl]])
                    soft_v[r, sl] = ex_v[r, sl] / d16

            scatter_add_rows(soft_v, egi_v, eg_sh)
            plsc.subcore_barrier()

            pltpu.sync_copy(eg_sh, egv)
            pltpu.sync_copy(dst_hbm.at[pl.ds(base, WROWS)], dst_v)

            @pl.loop(0, WROWS)
            def _(r):
                for c in range(8):
                    sl = pl.ds(c * 16, 16)
                    e16 = plsc.load_gather(egv, [egi_v[r, sl]])
                    ex_v[r, sl] = soft_v[r, sl] / e16

            scatter_add_rows(ex_v, dst_v, out_sh)
            plsc.subcore_barrier()

            pltpu.sync_copy(out_sh.at[pl.ds(w * ZCH, ZCH)], zbuf)
            pltpu.sync_copy(zbuf, out_hbm.at[pl.ds(w * ZCH, ZCH)])

    return k(att2, seg2, eg2, dst2)


def kernel(node_attention, memorized_embedding, rel_emb, query_src_emb,
           query_rel_emb, query_time_emb, eg_idx, idx_vi, idx_vj, seg_src,
           dst_ids, W_proj, b_proj, W_left, b_left, W_right, b_right,
           W_center, b_center):
    f32 = jnp.float32
    wl = [W_left[k * NSM:(k + 1) * NSM] for k in range(5)]
    wr = [W_right[k * NSM:(k + 1) * NSM] for k in range(5)]

    wm = jnp.concatenate([W_proj @ wl[0], W_proj @ wr[0]], axis=1)
    wrel = jnp.concatenate([W_proj @ wl[1], W_proj @ wr[1]], axis=1)

    q_src = query_src_emb @ W_proj + b_proj
    q_rel = query_rel_emb @ W_proj + b_proj
    q_time = query_time_emb @ W_proj + b_proj
    ql = (q_src @ wl[2] + q_rel @ wl[3] + q_time @ wl[4]
          + b_proj @ (wl[0] + wl[1]) + b_left)
    qr = (q_src @ wr[2] + q_rel @ wr[3] + q_time @ wr[4]
          + b_proj @ (wr[0] + wr[1]) + b_right)

    al, ar = _memproj(memorized_embedding, wm)
    pad = EP - E
    ivi = jnp.concatenate([idx_vi, jnp.zeros((pad,), jnp.int32)])
    ivj = jnp.concatenate([idx_vj, jnp.zeros((pad,), jnp.int32)])

    CUT = 51200
    bc2 = b_center.reshape(1, NSM)
    gl0, gr0 = _sc_gather(al, ar, ivi[:CUT], ivj[:CUT], CUT, 80)
    gl1, gr1 = _sc_gather(al, ar, ivi[CUT:], ivj[CUT:], EP - CUT, 96)

    eg0 = eg_idx[:CUT].reshape(32, 1, 1600)
    na0 = node_attention[:CUT].reshape(32, 1, 1600)
    att0 = _edge_stage(rel_emb, gl0, gr0, eg0, na0, ql, qr, wrel,
                       W_center, bc2, 1600, 32, 0)
    eg1 = eg_idx[CUT:].reshape(61, 1, 800)
    na1 = node_attention[CUT:].reshape(61, 1, 800)
    att1 = _edge_stage(rel_emb, gl1, gr1, eg1,
                       na1, ql, qr, wrel, W_center, bc2, 800, 61, 64)

    pad2 = EP2 - E
    att2 = jnp.concatenate(
        [att0.reshape(CUT), att1.reshape(E - CUT), jnp.zeros((pad2,), f32)])
    att2 = att2.reshape(EROWS, 128)
    seg2 = jnp.concatenate(
        [seg_src, jnp.full((pad2,), NSEG, jnp.int32)]).reshape(EROWS, 128)
    eg2 = jnp.concatenate(
        [eg_idx, jnp.full((pad2,), NB, jnp.int32)]).reshape(EROWS, 128)
    dst2 = jnp.concatenate(
        [dst_ids, jnp.full((pad2,), NTGT, jnp.int32)]).reshape(EROWS, 128)

    out_p = _sc_segment(att2, seg2, eg2, dst2)
    return out_p[:NTGT]


# speedup vs baseline: 1.2265x; 1.0156x over previous
import dataclasses
import functools

import jax
import jax.numpy as jnp
from jax import lax
from jax.experimental import pallas as pl
from jax.experimental.pallas import tpu as pltpu
from jax.experimental.pallas import tpu_sc as plsc

E = 100000
ND = 512
NSM = 128
NB = 64
NSEG = 25000
NTGT = 25000
MEMR = 40000

NC = 2
NS = 16
NW = NC * NS

EP = 100352
EP2 = 114688
EROWS = EP2 // 128
WROWS = EROWS // NS
SEG_PAD = 25088
OUT_PAD = 25088
EG_PAD = 80
ZCH = SEG_PAD // NS

GCH = EP // NW
GSTEP = 112
GN = GCH // GSTEP

_PREC = lax.Precision.DEFAULT

_MESH = plsc.VectorSubcoreMesh(core_axis_name="c", subcore_axis_name="s")

_SC_CP = pltpu.CompilerParams()
if "needs_layout_passes" in pltpu.CompilerParams.__dataclass_fields__:
    _SC_CP = dataclasses.replace(_SC_CP, needs_layout_passes=False)


def _memproj_body(x_ref, w_ref, ol_ref, or_ref):
    a = jnp.dot(x_ref[...], w_ref[...], precision=_PREC,
                preferred_element_type=jnp.float32)
    ol_ref[...] = a[:, :NSM]
    or_ref[...] = a[:, NSM:]


def _memproj(mem, wm):
    bm = 2000
    return pl.pallas_call(
        _memproj_body,
        grid=(MEMR // bm,),
        in_specs=[
            pl.BlockSpec((bm, ND), lambda i: (i, 0)),
            pl.BlockSpec((ND, 2 * NSM), lambda i: (0, 0)),
        ],
        out_specs=[
            pl.BlockSpec((bm, NSM), lambda i: (i, 0)),
            pl.BlockSpec((bm, NSM), lambda i: (i, 0)),
        ],
        out_shape=[
            jax.ShapeDtypeStruct((MEMR, NSM), jnp.float32),
            jax.ShapeDtypeStruct((MEMR, NSM), jnp.float32),
        ],
    )(mem, wm)


def _sc_gather(al, ar, ivi, ivj, nrows, gstep):
    gch = nrows // NW
    gn = gch // gstep

    assert gn % 2 == 0

    @functools.partial(
        pl.kernel,
        out_type=(
            jax.ShapeDtypeStruct((nrows, NSM), jnp.float32),
            jax.ShapeDtypeStruct((nrows, NSM), jnp.float32),
        ),
        mesh=_MESH,
        scratch_types=[
            pltpu.VMEM((gstep,), jnp.int32),
            pltpu.VMEM((gstep,), jnp.int32),
            pltpu.VMEM((gstep,), jnp.int32),
            pltpu.VMEM((gstep,), jnp.int32),
            pltpu.VMEM((gstep, NSM), jnp.float32),
            pltpu.VMEM((gstep, NSM), jnp.float32),
            pltpu.VMEM((gstep, NSM), jnp.float32),
            pltpu.VMEM((gstep, NSM), jnp.float32),
            pltpu.SemaphoreType.DMA,
            pltpu.SemaphoreType.DMA,
            pltpu.SemaphoreType.DMA,
            pltpu.SemaphoreType.DMA,
        ],
    )
    def k(al_hbm, ar_hbm, ivi_hbm, ivj_hbm, gl_hbm, gr_hbm,
          ii0, ij0, ii1, ij1, ri0, rj0, ri1, rj1, si0, sj0, si1, sj1):
        wid = lax.axis_index("s") * NC + lax.axis_index("c")
        base = wid * gch

        def start(ii, ij, ri, rj, si, sj, off):
            pltpu.sync_copy(ivi_hbm.at[pl.ds(off, gstep)], ii)
            pltpu.sync_copy(ivj_hbm.at[pl.ds(off, gstep)], ij)
            pltpu.async_copy(al_hbm.at[ii], ri, si)
            pltpu.async_copy(ar_hbm.at[ij], rj, sj)

        def finish(ii, ij, ri, rj, si, sj, off):
            pltpu.make_async_copy(al_hbm.at[ii], ri, si).wait()
            pltpu.make_async_copy(ar_hbm.at[ij], rj, sj).wait()
            pltpu.sync_copy(ri, gl_hbm.at[pl.ds(off, gstep)])
            pltpu.sync_copy(rj, gr_hbm.at[pl.ds(off, gstep)])

        start(ii0, ij0, ri0, rj0, si0, sj0, base)

        @pl.loop(0, gn // 2)
        def _(g):
            j0 = base + 2 * g * gstep
            start(ii1, ij1, ri1, rj1, si1, sj1, j0 + gstep)
            finish(ii0, ij0, ri0, rj0, si0, sj0, j0)

            @pl.when(2 * g + 2 < gn)
            def _():
                start(ii0, ij0, ri0, rj0, si0, sj0, j0 + 2 * gstep)

            finish(ii1, ij1, ri1, rj1, si1, sj1, j0 + gstep)

    return k(al, ar, ivi, ivj)


def _edge_body(rel_ref, gl_ref, gr_ref, eg_ref, na_ref, ql_ref, qr_ref,
               wrel_ref, wc_ref, bc_ref, att_ref):
    r2 = jnp.dot(rel_ref[...], wrel_ref[...], precision=_PREC,
                 preferred_element_type=jnp.float32)
    eg = eg_ref[0, 0, :]
    onehot = (eg[:, None] ==
              lax.broadcasted_iota(jnp.int32, (1, NB), 1)).astype(jnp.float32)
    qlg = jnp.dot(onehot, ql_ref[...], precision=_PREC,
                  preferred_element_type=jnp.float32)
    qrg = jnp.dot(onehot, qr_ref[...], precision=_PREC,
                  preferred_element_type=jnp.float32)
    left = r2[:, :NSM] + gl_ref[...] + qlg
    right = r2[:, NSM:] + gr_ref[...] + qrg
    lh = jnp.where(left >= 0, left, 0.01 * left)
    rh = jnp.where(right >= 0, right, 0.01 * right)
    ch = jnp.dot(rh, wc_ref[...], precision=_PREC,
                 preferred_element_type=jnp.float32) + bc_ref[...]
    logits = jnp.sum(lh * ch, axis=1)
    att_ref[0, 0, :] = logits * na_ref[0, 0, :]


def _edge_stage(rel, gl, gr, eg3, na3, ql, qr, wrel, wc, bc2, be, ng, offb):
    return pl.pallas_call(
        _edge_body,
        grid=(ng,),
        in_specs=[
            pl.BlockSpec((be, ND), lambda i: (i + offb, 0)),
            pl.BlockSpec((be, NSM), lambda i: (i, 0)),
            pl.BlockSpec((be, NSM), lambda i: (i, 0)),
            pl.BlockSpec((1, 1, be), lambda i: (i, 0, 0)),
            pl.BlockSpec((1, 1, be), lambda i: (i, 0, 0)),
            pl.BlockSpec((NB, NSM), lambda i: (0, 0)),
            pl.BlockSpec((NB, NSM), lambda i: (0, 0)),
            pl.BlockSpec((ND, 2 * NSM), lambda i: (0, 0)),
            pl.BlockSpec((NSM, NSM), lambda i: (0, 0)),
            pl.BlockSpec((1, NSM), lambda i: (0, 0)),
        ],
        out_specs=pl.BlockSpec((1, 1, be), lambda i: (i, 0, 0)),
        out_shape=jax.ShapeDtypeStruct((ng, 1, be), jnp.float32),
    )(rel, gl, gr, eg3, na3, ql, qr, wrel, wc, bc2)


def _sc_segment(att2, seg2, eg2, dst2):
    @functools.partial(
        pl.kernel,
        out_type=jax.ShapeDtypeStruct((OUT_PAD,), jnp.float32),
        mesh=_MESH,
        compiler_params=_SC_CP,
        scratch_types=[
            pltpu.VMEM_SHARED((SEG_PAD,), jnp.float32),
            pltpu.VMEM_SHARED((EG_PAD,), jnp.float32),
            pltpu.VMEM_SHARED((OUT_PAD,), jnp.float32),
            pltpu.VMEM_SHARED((NS, 16), jnp.float32),
            pltpu.VMEM((ZCH,), jnp.float32),
            pltpu.VMEM((WROWS, 128), jnp.float32),
            pltpu.VMEM((WROWS, 128), jnp.float32),
            pltpu.VMEM((WROWS, 128), jnp.float32),
            pltpu.VMEM((WROWS, 128), jnp.int32),
            pltpu.VMEM((WROWS, 128), jnp.int32),
            pltpu.VMEM((WROWS, 128), jnp.int32),
            pltpu.VMEM((SEG_PAD,), jnp.float32),
            pltpu.VMEM((EG_PAD,), jnp.float32),
            pltpu.VMEM((EG_PAD,), jnp.float32),
            pltpu.VMEM((EG_PAD,), jnp.int32),
            pltpu.VMEM((16,), jnp.float32),
            pltpu.VMEM((NS, 16), jnp.float32),
            pltpu.SemaphoreType.DMA,
        ],
    )
    def k(att_hbm, seg_hbm, eg_hbm, dst_hbm, out_hbm,
          den_sh, eg_sh, out_sh, max_sh, zbuf, att_v, ex_v, soft_v,
          segi_v, egi_v, dst_v, den_v, egv, egloc, idn, gmax_v, slab_v, ssem):
        core = lax.axis_index("c")

        def scatter_add_rows(val_v, idx_v, acc_sh):
            @pl.loop(0, WROWS // 7)
            def _(g):
                for u in range(7):
                    r = g * 7 + u
                    pltpu.async_copy(val_v.at[r], acc_sh.at[idx_v.at[r]],
                                     ssem, add=True)
                for u in range(7):
                    r = g * 7 + u
                    pltpu.make_async_copy(val_v.at[r],
                                          acc_sh.at[idx_v.at[r]],
                                          ssem).wait()

        @pl.when(core == 0)
        def _():
            w = lax.axis_index("s")
            base = w * WROWS

            @pl.loop(0, ZCH, step=16)
            def _(i):
                zbuf[pl.ds(i, 16)] = jnp.zeros((16,), jnp.float32)

            pltpu.sync_copy(zbuf, den_sh.at[pl.ds(w * ZCH, ZCH)])
            pltpu.sync_copy(zbuf, out_sh.at[pl.ds(w * ZCH, ZCH)])

            @pl.when(w == 0)
            def _():
                pltpu.sync_copy(zbuf.at[pl.ds(0, EG_PAD)], eg_sh)

            pltpu.sync_copy(att_hbm.at[pl.ds(base, WROWS)], att_v)
            pltpu.sync_copy(seg_hbm.at[pl.ds(base, WROWS)], segi_v)

            gmax_v[...] = att_v[0, pl.ds(0, 16)]

            @pl.loop(0, WROWS)
            def _(r):
                for c in range(8):
                    gmax_v[...] = jnp.maximum(gmax_v[...],
                                              att_v[r, pl.ds(c * 16, 16)])

            mloc = lax.reduce_max(gmax_v[...], axes=(0,))
            gmax_v[...] = jnp.full((16,), mloc, jnp.float32)
            pltpu.sync_copy(gmax_v, max_sh.at[w])
            plsc.subcore_barrier()

            pltpu.sync_copy(max_sh, slab_v)
            gmax_v[...] = slab_v[0, pl.ds(0, 16)]
            for t in range(1, NS):
                gmax_v[...] = jnp.maximum(gmax_v[...], slab_v[t, pl.ds(0, 16)])
            gm = jnp.full((16,), lax.reduce_max(gmax_v[...], axes=(0,)),
                          jnp.float32)

            @pl.loop(0, WROWS)
            def _(r):
                for c in range(8):
                    sl = pl.ds(c * 16, 16)
                    ex_v[r, sl] = jnp.exp(att_v[r, sl] - gm)

            scatter_add_rows(ex_v, segi_v, den_sh)
            plsc.subcore_barrier()

            pltpu.sync_copy(den_sh, den_v)
            pltpu.sync_copy(eg_hbm.at[pl.ds(base, WROWS)], egi_v)
            for c in range(EG_PAD // 16):
                sl = pl.ds(c * 16, 16)
                egloc[sl] = jnp.zeros((16,), jnp.float32)
                idn[sl] = lax.iota(jnp.int32, 16) + c * 16

            @pl.loop(0, WROWS)
            def _(r):
                for c in range(8):
                    sl = pl.ds(c * 16, 16)
                    d16 = plsc.load_gather(den_v, [segi_v[r, sl]])
                    s16 = ex_v[r, sl] / d16
                    soft_v[r, sl] = s16
                    plsc.addupdate_scatter(egloc, [egi_v[r, sl]], s16)

            pltpu.async_copy(egloc, eg_sh.at[idn], ssem, add=True).wait()
            plsc.subcore_barrier()

            pltpu.sync_copy(eg_sh, egv)
            pltpu.sync_copy(dst_hbm.at[pl.ds(base, WROWS)], dst_v)

            @pl.loop(0, WROWS)
            def _(r):
                for c in range(8):
                    sl = pl.ds(c * 16, 16)
                    e16 = plsc.load_gather(egv, [egi_v[r, sl]])
                    ex_v[r, sl] = soft_v[r, sl] / e16

            scatter_add_rows(ex_v, dst_v, out_sh)
            plsc.subcore_barrier()

            pltpu.sync_copy(out_sh.at[pl.ds(w * ZCH, ZCH)], zbuf)
            pltpu.sync_copy(zbuf, out_hbm.at[pl.ds(w * ZCH, ZCH)])

    return k(att2, seg2, eg2, dst2)


def kernel(node_attention, memorized_embedding, rel_emb, query_src_emb,
           query_rel_emb, query_time_emb, eg_idx, idx_vi, idx_vj, seg_src,
           dst_ids, W_proj, b_proj, W_left, b_left, W_right, b_right,
           W_center, b_center):
    f32 = jnp.float32
    wl = [W_left[k * NSM:(k + 1) * NSM] for k in range(5)]
    wr = [W_right[k * NSM:(k + 1) * NSM] for k in range(5)]

    wm = jnp.concatenate([W_proj @ wl[0], W_proj @ wr[0]], axis=1)
    wrel = jnp.concatenate([W_proj @ wl[1], W_proj @ wr[1]], axis=1)

    q_src = query_src_emb @ W_proj + b_proj
    q_rel = query_rel_emb @ W_proj + b_proj
    q_time = query_time_emb @ W_proj + b_proj
    ql = (q_src @ wl[2] + q_rel @ wl[3] + q_time @ wl[4]
          + b_proj @ (wl[0] + wl[1]) + b_left)
    qr = (q_src @ wr[2] + q_rel @ wr[3] + q_time @ wr[4]
          + b_proj @ (wr[0] + wr[1]) + b_right)

    al, ar = _memproj(memorized_embedding, wm)
    pad = EP - E
    ivi = jnp.concatenate([idx_vi, jnp.zeros((pad,), jnp.int32)])
    ivj = jnp.concatenate([idx_vj, jnp.zeros((pad,), jnp.int32)])

    CUT = 51200
    bc2 = b_center.reshape(1, NSM)
    gl0, gr0 = _sc_gather(al, ar, ivi[:CUT], ivj[:CUT], CUT, 80)
    gl1, gr1 = _sc_gather(al, ar, ivi[CUT:], ivj[CUT:], EP - CUT, 96)

    eg0 = eg_idx[:CUT].reshape(32, 1, 1600)
    na0 = node_attention[:CUT].reshape(32, 1, 1600)
    att0 = _edge_stage(rel_emb, gl0, gr0, eg0, na0, ql, qr, wrel,
                       W_center, bc2, 1600, 32, 0)
    eg1 = eg_idx[CUT:].reshape(61, 1, 800)
    na1 = node_attention[CUT:].reshape(61, 1, 800)
    att1 = _edge_stage(rel_emb, gl1, gr1, eg1,
                       na1, ql, qr, wrel, W_center, bc2, 800, 61, 64)

    pad2 = EP2 - E
    att2 = jnp.concatenate(
        [att0.reshape(CUT), att1.reshape(E - CUT), jnp.zeros((pad2,), f32)])
    att2 = att2.reshape(EROWS, 128)
    seg2 = jnp.concatenate(
        [seg_src, jnp.full((pad2,), NSEG, jnp.int32)]).reshape(EROWS, 128)
    eg2 = jnp.concatenate(
        [eg_idx, jnp.full((pad2,), NB, jnp.int32)]).reshape(EROWS, 128)
    dst2 = jnp.concatenate(
        [dst_ids, jnp.full((pad2,), NTGT, jnp.int32)]).reshape(EROWS, 128)

    out_p = _sc_segment(att2, seg2, eg2, dst2)
    return out_p[:NTGT]
